# baseline (speedup 1.0000x reference)
import functools

import jax
import jax.numpy as jnp
from jax import lax
from jax.experimental import pallas as pl
from jax.experimental.pallas import tpu as pltpu
from jax.experimental.pallas import tpu_sc as plsc

_B = 16384
_F = 26
_V = 2
_D = 128

_NC = 2
_NS = 16
_NW = _NC * _NS

_N = _B * _F
_PER_W = _N // _NW
_CH = 64
_NCH = _PER_W // _CH


def _lookup(xf, tabflat):
    mesh = plsc.VectorSubcoreMesh(core_axis_name="c", subcore_axis_name="s")

    @functools.partial(
        pl.kernel,
        out_type=jax.ShapeDtypeStruct((_N, _D), jnp.float32),
        mesh=mesh,
        scratch_types=[
            pltpu.VMEM((_PER_W,), jnp.int32),
            pltpu.VMEM((_F * _V * _D,), jnp.float32),
            pltpu.VMEM((2, _CH, _D), jnp.float32),
            pltpu.SemaphoreType.DMA,
            pltpu.SemaphoreType.DMA,
        ],
    )
    def body(xf_hbm, tab_hbm, out_hbm, xall, tabv, rows, s0, s1):
        osem = (s0, s1)
        wid = lax.axis_index("s") * _NC + lax.axis_index("c")
        wbase = wid * _PER_W
        pltpu.sync_copy(tab_hbm, tabv)
        pltpu.sync_copy(xf_hbm.at[pl.ds(wbase, _PER_W)], xall)

        lanes = lax.iota(jnp.int32, 16)

        def build_chunk(j, b):
            base = j * _CH
            for g in range(_CH // 16):
                off = base + g * 16
                pos = (wbase + off) + lanes
                f = lax.rem(pos, _F)
                iv = (xall[pl.ds(off, 16)] + 2 * f) * _D
                for l in range(16):
                    roff = iv[l]
                    r = g * 16 + l
                    for c in range(_D // 16):
                        rows[b, r, pl.ds(c * 16, 16)] = (
                            tabv[pl.ds(roff + c * 16, 16)])

        def fire_scatter(j, b):
            pltpu.async_copy(
                rows.at[b], out_hbm.at[pl.ds(wbase + j * _CH, _CH)], osem[b])

        def wait_scatter(b):
            pltpu.make_async_copy(
                rows.at[b], out_hbm.at[pl.ds(wbase, _CH)], osem[b]).wait()

        build_chunk(0, 0)
        fire_scatter(0, 0)
        build_chunk(1, 1)
        fire_scatter(1, 1)

        def outer(s, carry):
            for k in range(2):
                j = 2 + s * 2 + k
                b = k
                wait_scatter(b)
                build_chunk(j, b)
                fire_scatter(j, b)
            return carry

        lax.fori_loop(0, (_NCH - 2) // 2, outer, 0)

        wait_scatter(0)
        wait_scatter(1)

    return body(xf, tabflat)


def kernel(X, tables):
    xf = X.reshape(_N)
    tabflat = tables.reshape(_F * _V * _D)
    out = _lookup(xf, tabflat)
    return out.reshape(_B, _F, 1, _D)
